# Initial kernel scaffold; baseline (speedup 1.0000x reference)
import functools

import jax
import jax.numpy as jnp
from jax import lax
from jax.experimental import pallas as pl
from jax.experimental.pallas import tpu as pltpu
from jax.experimental.pallas import tpu_sc as plsc

_NUM_CORES = 2
_NUM_SUBCORES = 16
_LANES = 16
_NUM_WORKERS = _NUM_CORES * _NUM_SUBCORES


def _make_kernel(B, L, V, D):
    rows_per_w = B // _NUM_WORKERS
    g0 = min(L, 128)
    g1 = L - g0
    kvecs = D // _LANES

    mesh = plsc.VectorSubcoreMesh(
        core_axis_name="c",
        subcore_axis_name="s",
        num_cores=_NUM_CORES,
        num_subcores=_NUM_SUBCORES,
    )

    @functools.partial(
        pl.kernel,
        out_type=jax.ShapeDtypeStruct((B * L, D), jnp.float32),
        mesh=mesh,
        scratch_types=[
            pltpu.VMEM((L,), jnp.int32),
            pltpu.VMEM((L, D), jnp.float32),
            pltpu.VMEM((L, D), jnp.float32),
            pltpu.VMEM((1, D), jnp.float32),
            pltpu.SemaphoreType.DMA,
        ],
    )
    def emb_kernel(tok_hbm, emb_hbm, pos_hbm, seg_hbm, out_hbm,
                   idx_v, rows_v, bias_v, seg_v, sem):
        wid = lax.axis_index("s") * _NUM_CORES + lax.axis_index("c")

        pltpu.sync_copy(pos_hbm.at[pl.ds(0, L)], bias_v)
        pltpu.sync_copy(seg_hbm.at[pl.ds(0, 1)], seg_v)

        @pl.loop(0, L)
        def _(l):
            for k in range(kvecs):
                s = seg_v[0, pl.ds(k * _LANES, _LANES)]
                plsc.addupdate(bias_v.at[l, pl.ds(k * _LANES, _LANES)], s)

        base_row = wid * rows_per_w

        @pl.loop(0, rows_per_w)
        def _(r):
            tok_off = (base_row + r) * L
            pltpu.sync_copy(tok_hbm.at[pl.ds(tok_off, L)], idx_v)
            cp0 = pltpu.async_copy(
                emb_hbm.at[idx_v.at[pl.ds(0, g0)]],
                rows_v.at[pl.ds(0, g0)], sem)
            if g1:
                cp1 = pltpu.async_copy(
                    emb_hbm.at[idx_v.at[pl.ds(g0, g1)]],
                    rows_v.at[pl.ds(g0, g1)], sem)
            cp0.wait()
            if g1:
                cp1.wait()

            @pl.loop(0, L)
            def _(l):
                for k in range(kvecs):
                    b = bias_v[l, pl.ds(k * _LANES, _LANES)]
                    plsc.addupdate(rows_v.at[l, pl.ds(k * _LANES, _LANES)], b)

            pltpu.sync_copy(rows_v, out_hbm.at[pl.ds(tok_off, L)])

    return emb_kernel


@jax.jit
def kernel(tokens, Embead, PosEmbead, SegEmbead):
    B, L = tokens.shape
    V, D = Embead.shape
    tok_flat = tokens.reshape(B * L).astype(jnp.int32)
    emb_k = _make_kernel(B, L, V, D)
    out = emb_k(tok_flat, Embead, PosEmbead, SegEmbead)
    return out.reshape(B, L, D)


# speedup vs baseline: 6.9905x; 6.9905x over previous
import functools

import jax
import jax.numpy as jnp
from jax import lax
from jax.experimental import pallas as pl
from jax.experimental.pallas import tpu as pltpu
from jax.experimental.pallas import tpu_sc as plsc

_NUM_CORES = 2
_NUM_SUBCORES = 16
_LANES = 16
_NUM_WORKERS = _NUM_CORES * _NUM_SUBCORES


def _make_kernel(B, L, V, D):
    rows_per_w = B // _NUM_WORKERS
    g0 = min(L, 128)
    g1 = L - g0
    kvecs = D // _LANES

    mesh = plsc.VectorSubcoreMesh(
        core_axis_name="c",
        subcore_axis_name="s",
        num_cores=_NUM_CORES,
        num_subcores=_NUM_SUBCORES,
    )

    @functools.partial(
        pl.kernel,
        out_type=jax.ShapeDtypeStruct((B * L, D), jnp.float32),
        mesh=mesh,
        compiler_params=pltpu.CompilerParams(use_tc_tiling_on_sc=False),
        scratch_types=[
            pltpu.VMEM((L,), jnp.int32),
            pltpu.VMEM((L, D), jnp.float32),
            pltpu.VMEM((L, D), jnp.float32),
            pltpu.VMEM((1, D), jnp.float32),
            pltpu.SemaphoreType.DMA,
        ],
    )
    def emb_kernel(tok_hbm, emb_hbm, pos_hbm, seg_hbm, out_hbm,
                   idx_v, rows_v, bias_v, seg_v, sem):
        wid = lax.axis_index("s") * _NUM_CORES + lax.axis_index("c")

        pltpu.sync_copy(pos_hbm.at[pl.ds(0, L)], bias_v)
        pltpu.sync_copy(seg_hbm.at[pl.ds(0, 1)], seg_v)

        @pl.loop(0, L)
        def _(l):
            for k in range(kvecs):
                s = seg_v[0, pl.ds(k * _LANES, _LANES)]
                plsc.addupdate(bias_v.at[l, pl.ds(k * _LANES, _LANES)], s)

        base_row = wid * rows_per_w

        @pl.loop(0, rows_per_w)
        def _(r):
            tok_off = (base_row + r) * L
            pltpu.sync_copy(tok_hbm.at[pl.ds(tok_off, L)], idx_v)
            cp0 = pltpu.async_copy(
                emb_hbm.at[idx_v.at[pl.ds(0, g0)]],
                rows_v.at[pl.ds(0, g0)], sem)
            if g1:
                cp1 = pltpu.async_copy(
                    emb_hbm.at[idx_v.at[pl.ds(g0, g1)]],
                    rows_v.at[pl.ds(g0, g1)], sem)
            cp0.wait()
            if g1:
                cp1.wait()

            @pl.loop(0, L)
            def _(l):
                for k in range(kvecs):
                    b = bias_v[l, pl.ds(k * _LANES, _LANES)]
                    plsc.addupdate(rows_v.at[l, pl.ds(k * _LANES, _LANES)], b)

            pltpu.sync_copy(rows_v, out_hbm.at[pl.ds(tok_off, L)])

    return emb_kernel


@jax.jit
def kernel(tokens, Embead, PosEmbead, SegEmbead):
    B, L = tokens.shape
    V, D = Embead.shape
    tok_flat = tokens.reshape(B * L).astype(jnp.int32)
    emb_k = _make_kernel(B, L, V, D)
    out = emb_k(tok_flat, Embead, PosEmbead, SegEmbead)
    return out.reshape(B, L, D)


# speedup vs baseline: 8.6615x; 1.2390x over previous
import functools

import jax
import jax.numpy as jnp
from jax import lax
from jax.experimental import pallas as pl
from jax.experimental.pallas import tpu as pltpu
from jax.experimental.pallas import tpu_sc as plsc

_NUM_CORES = 2
_NUM_SUBCORES = 16
_LANES = 16
_NUM_WORKERS = _NUM_CORES * _NUM_SUBCORES
_NBUF = 4


def _make_kernel(B, L, V, D):
    rows_per_w = B // _NUM_WORKERS
    g0 = min(L, 128)
    g1 = L - g0
    kvecs = D // _LANES

    mesh = plsc.VectorSubcoreMesh(
        core_axis_name="c",
        subcore_axis_name="s",
        num_cores=_NUM_CORES,
        num_subcores=_NUM_SUBCORES,
    )

    @functools.partial(
        pl.kernel,
        out_type=jax.ShapeDtypeStruct((B * L, D), jnp.float32),
        mesh=mesh,
        compiler_params=pltpu.CompilerParams(use_tc_tiling_on_sc=False),
        scratch_types=[
            pltpu.VMEM((rows_per_w * L,), jnp.int32),
            [pltpu.VMEM((L, D), jnp.float32) for _ in range(_NBUF)],
            pltpu.VMEM((L, D), jnp.float32),
            pltpu.VMEM((1, D), jnp.float32),
            [pltpu.SemaphoreType.DMA for _ in range(_NBUF)],
        ],
    )
    def emb_kernel(tok_hbm, emb_hbm, pos_hbm, seg_hbm, out_hbm,
                   idx_all, rows, bias_v, seg_v, sems):
        wid = lax.axis_index("s") * _NUM_CORES + lax.axis_index("c")
        base = wid * (rows_per_w * L)

        pltpu.sync_copy(tok_hbm.at[pl.ds(base, rows_per_w * L)], idx_all)

        pltpu.sync_copy(pos_hbm.at[pl.ds(0, L)], bias_v)
        pltpu.sync_copy(seg_hbm.at[pl.ds(0, 1)], seg_v)

        @pl.loop(0, L, unroll=4)
        def _(l):
            for k in range(kvecs):
                s = seg_v[0, pl.ds(k * _LANES, _LANES)]
                plsc.addupdate(bias_v.at[l, pl.ds(k * _LANES, _LANES)], s)

        def issue_gather(r):
            b = r % _NBUF
            off = r * L
            cps = [pltpu.async_copy(
                emb_hbm.at[idx_all.at[pl.ds(off, g0)]],
                rows[b].at[pl.ds(0, g0)], sems[b])]
            if g1:
                cps.append(pltpu.async_copy(
                    emb_hbm.at[idx_all.at[pl.ds(off + g0, g1)]],
                    rows[b].at[pl.ds(g0, g1)], sems[b]))
            return cps

        def issue_write(r):
            b = r % _NBUF
            return pltpu.async_copy(
                rows[b], out_hbm.at[pl.ds(base + r * L, L)], sems[b])

        gathers = {}
        writes = {}
        for r in range(min(2, rows_per_w)):
            gathers[r] = issue_gather(r)

        for r in range(rows_per_w):
            b = r % _NBUF
            for cp in gathers.pop(r):
                cp.wait()
            if r - 2 in writes:
                writes.pop(r - 2).wait()
            if r + 2 < rows_per_w:
                gathers[r + 2] = issue_gather(r + 2)

            @pl.loop(0, L, unroll=4)
            def _(l):
                for k in range(kvecs):
                    v = bias_v[l, pl.ds(k * _LANES, _LANES)]
                    plsc.addupdate(rows[b].at[l, pl.ds(k * _LANES, _LANES)], v)

            writes[r] = issue_write(r)

        for r in sorted(writes):
            writes.pop(r).wait()

    return emb_kernel


@jax.jit
def kernel(tokens, Embead, PosEmbead, SegEmbead):
    B, L = tokens.shape
    V, D = Embead.shape
    tok_flat = tokens.reshape(B * L).astype(jnp.int32)
    emb_k = _make_kernel(B, L, V, D)
    out = emb_k(tok_flat, Embead, PosEmbead, SegEmbead)
    return out.reshape(B, L, D)


# speedup vs baseline: 25.6244x; 2.9584x over previous
import functools

import jax
import jax.numpy as jnp
from jax import lax
from jax.experimental import pallas as pl
from jax.experimental.pallas import tpu as pltpu
from jax.experimental.pallas import tpu_sc as plsc

_NUM_CORES = 2
_NUM_SUBCORES = 16
_LANES = 16
_NUM_WORKERS = _NUM_CORES * _NUM_SUBCORES
_CP = 4
_KB = 16


def _make_kernel(B, L, V, D, P):
    d_per_w = D // _NUM_WORKERS
    nch = L // _CP
    nb = B // _LANES
    lpad = -(-L // _LANES) * _LANES

    mesh = plsc.VectorSubcoreMesh(
        core_axis_name="c",
        subcore_axis_name="s",
        num_cores=_NUM_CORES,
        num_subcores=_NUM_SUBCORES,
    )

    @functools.partial(
        pl.kernel,
        out_type=jax.ShapeDtypeStruct(
            (L, D // 8, B // 128, 8, 128), jnp.float32),
        mesh=mesh,
        compiler_params=pltpu.CompilerParams(
            use_tc_tiling_on_sc=True, needs_layout_passes=False),
        scratch_types=[
            pltpu.VMEM((1, V), jnp.float32),
            [pltpu.VMEM((_CP * B,), jnp.int32) for _ in range(2)],
            [pltpu.VMEM((_CP, 1, B // 128, 1, 128), jnp.float32)
             for _ in range(2)],
            pltpu.VMEM((lpad,), jnp.float32),
            pltpu.VMEM((lpad,), jnp.float32),
            pltpu.VMEM((D,), jnp.float32),
            pltpu.VMEM_SHARED((L * B,), jnp.int32),
            [pltpu.SemaphoreType.DMA for _ in range(2)],
            [pltpu.SemaphoreType.DMA for _ in range(2)],
            pltpu.SemaphoreType.DMA,
        ],
    )
    def emb_kernel(tok_hbm, emb_hbm, pos_hbm, seg_hbm, out_hbm,
                   row_v, tok_bufs, out_bufs, pos_v, bias_v, seg_v,
                   tok_sh, tsems, osems, rsem):
        wid = lax.axis_index("s") * _NUM_CORES + lax.axis_index("c")
        zeros16 = jnp.zeros((_LANES,), jnp.int32)

        @pl.when(lax.axis_index("s") == 0)
        def _():
            pltpu.sync_copy(tok_hbm, tok_sh)

        pltpu.sync_copy(seg_hbm.at[pl.ds(0, D)], seg_v)
        plsc.subcore_barrier()

        def tok_copy(c, b):
            return pltpu.make_async_copy(
                tok_sh.at[pl.ds(c * (_CP * B), _CP * B)], tok_bufs[b],
                tsems[b])

        def out_copy(c, b, d):
            return pltpu.make_async_copy(
                out_bufs[b],
                out_hbm.at[pl.ds(c * _CP, _CP), pl.ds(d // 8, 1), :,
                           pl.ds(d % 8, 1), :], osems[b])

        def compute(c, b):
            @pl.loop(0, _CP)
            def _(i):
                l = c * _CP + i
                bias = plsc.load_gather(
                    bias_v, [jnp.full((_LANES,), l, jnp.int32)])

                @pl.loop(0, nb // _KB)
                def _(jo):
                    base = i * B + jo * _KB * _LANES
                    idxs = [
                        tok_bufs[b][pl.ds(base + k * _LANES, _LANES)]
                        for k in range(_KB)
                    ]
                    vals = [
                        plsc.load_gather(row_v, [zeros16, ix]) for ix in idxs
                    ]
                    for k in range(_KB):
                        bt = (jo * _KB + k) // 8
                        out_bufs[b][i, 0, bt, 0,
                                    pl.ds((k % 8) * _LANES, _LANES)] = (
                                        vals[k] + bias)

        @pl.loop(0, d_per_w)
        def _(di):
            d = wid * d_per_w + di
            row_cp = pltpu.make_async_copy(
                emb_hbm.at[pl.ds(d, 1), :], row_v, rsem)
            row_cp.start()
            tok_copy(0, 0).start()
            tok_copy(1, 1).start()
            pltpu.sync_copy(pos_hbm.at[pl.ds(d * P, lpad)], pos_v)
            seg_s = plsc.load_gather(
                seg_v, [jnp.full((_LANES,), d, jnp.int32)])
            for v in range(lpad // _LANES):
                sl = pl.ds(v * _LANES, _LANES)
                bias_v[sl] = pos_v[sl] + seg_s
            row_cp.wait()

            def run_chunk(c, b, first, last):
                tok_copy(c, b).wait()
                if not first:
                    out_copy(c - 2, b, d).wait()
                compute(c, b)
                out_copy(c, b, d).start()
                if not last:
                    tok_copy(c + 2, b).start()

            run_chunk(0, 0, True, False)
            run_chunk(1, 1, True, False)

            @pl.loop(2, nch - 2, step=2)
            def _(c0):
                run_chunk(c0, 0, False, False)
                run_chunk(c0 + 1, 1, False, False)

            run_chunk(nch - 2, 0, False, True)
            run_chunk(nch - 1, 1, False, True)
            out_copy(nch - 2, 0, d).wait()
            out_copy(nch - 1, 1, d).wait()

    return emb_kernel


@jax.jit
def kernel(tokens, Embead, PosEmbead, SegEmbead):
    B, L = tokens.shape
    V, D = Embead.shape
    P = PosEmbead.shape[0]
    emb_k = _make_kernel(B, L, V, D, P)
    out5 = emb_k(
        tokens.T.reshape(L * B).astype(jnp.int32),
        Embead.T,
        PosEmbead.T.reshape(D * P),
        SegEmbead.reshape(-1),
    )
    return out5.transpose(2, 4, 0, 1, 3).reshape(B, L, D)
